# Initial kernel scaffold; baseline (speedup 1.0000x reference)
import functools

import jax
import jax.numpy as jnp
from jax import lax
from jax.experimental import pallas as pl
from jax.experimental.pallas import tpu as pltpu
from jax.experimental.pallas import tpu_sc as plsc

NC = 2
NS = 16
NW = NC * NS
L = 16

B = 4096
D = 64
HIST = 50
PB = B // NW
CH = 128
NCH = PB * HIST // CH


def _wid():
    return lax.axis_index("s") * NC + lax.axis_index("c")


def _gather3_body(uids, iids, uw_t, iw_t, fb_t, uw_o, iw_o, fb_o,
                  uid_v, iid_v, uw_v, iw_v, fb_v, s0, s1, s2):
    base = pl.multiple_of(_wid() * PB, PB)
    pltpu.sync_copy(uids.at[pl.ds(base, PB)], uid_v)
    pltpu.sync_copy(iids.at[pl.ds(base, PB)], iid_v)
    d0 = pltpu.async_copy(uw_t.at[uid_v], uw_v, s0)
    d1 = pltpu.async_copy(iw_t.at[iid_v], iw_v, s1)
    d2 = pltpu.async_copy(fb_t.at[uid_v], fb_v, s2)
    d0.wait()
    pltpu.sync_copy(uw_v, uw_o.at[pl.ds(base, PB)])
    d1.wait()
    pltpu.sync_copy(iw_v, iw_o.at[pl.ds(base, PB)])
    d2.wait()
    pltpu.sync_copy(fb_v, fb_o.at[pl.ds(base, PB)])


def _implicit_body(fb_flat, ew_t, uw_in, uf_o,
                   idx_v, uw_v, uf_v, r0, r1, s0, s1):
    base = pl.multiple_of(_wid() * PB, PB)
    fb_base = pl.multiple_of(_wid() * (PB * HIST), PB * HIST)
    pltpu.sync_copy(fb_flat.at[pl.ds(fb_base, PB * HIST)], idx_v)
    pltpu.sync_copy(uw_in.at[pl.ds(base, PB)], uw_v)

    zero = jnp.zeros((L,), jnp.float32)

    def zbody(r, carry):
        for k in range(D // L):
            uf_v[r, pl.ds(k * L, L)] = zero
        return carry

    lax.fori_loop(0, PB, zbody, 0)

    rows = (r0, r1)
    sems = (s0, s1)

    def issue(c, buf):
        return pltpu.async_copy(
            ew_t.at[idx_v.at[pl.ds(c * CH, CH)]], rows[buf], sems[buf])

    descs = {0: issue(0, 0), 1: issue(1, 1)}
    for c in range(NCH):
        buf = c & 1
        descs[c].wait()
        rbuf = rows[buf]

        def abody(j, carry, c=c, rbuf=rbuf):
            t = (c * CH + j) // HIST
            for k in range(D // L):
                v = rbuf[j, pl.ds(k * L, L)]
                plsc.addupdate(uf_v.at[t, pl.ds(k * L, L)], v)
            return carry

        lax.fori_loop(0, CH, abody, 0)
        if c + 2 < NCH:
            descs[c + 2] = issue(c + 2, buf)

    inv = jnp.float32(1.0 / HIST)

    def fbody(r, carry):
        for k in range(D // L):
            sl = pl.ds(k * L, L)
            uf_v[r, sl] = uw_v[r, sl] + uf_v[r, sl] * inv
        return carry

    lax.fori_loop(0, PB, fbody, 0)
    pltpu.sync_copy(uf_v, uf_o.at[pl.ds(base, PB)])


def _mm_body(uf_ref, iw_ref, bias_ref, o_ref):
    acc = lax.dot_general(uf_ref[...], iw_ref[...],
                          (((1,), (1,)), ((), ())),
                          preferred_element_type=jnp.float32)
    x = acc + bias_ref[0]
    o_ref[...] = 1.0 / (1.0 + jnp.exp(-x))


def kernel(user_ids, item_ids, user_weight, item_weight,
           item_rating_effect_weight, bias, fb_indices):
    mesh = plsc.VectorSubcoreMesh(core_axis_name="c", subcore_axis_name="s",
                                  num_cores=NC, num_subcores=NS)

    gather3 = pl.kernel(
        _gather3_body,
        out_type=(
            jax.ShapeDtypeStruct((B, D), jnp.float32),
            jax.ShapeDtypeStruct((B, D), jnp.float32),
            jax.ShapeDtypeStruct((B, HIST), jnp.int32),
        ),
        mesh=mesh,
        scratch_types=[
            pltpu.VMEM((PB,), jnp.int32),
            pltpu.VMEM((PB,), jnp.int32),
            pltpu.VMEM((PB, D), jnp.float32),
            pltpu.VMEM((PB, D), jnp.float32),
            pltpu.VMEM((PB, HIST), jnp.int32),
            pltpu.SemaphoreType.DMA,
            pltpu.SemaphoreType.DMA,
            pltpu.SemaphoreType.DMA,
        ],
    )
    uw, iw, batch_fb = gather3(user_ids, item_ids, user_weight, item_weight,
                               fb_indices)

    implicit = pl.kernel(
        _implicit_body,
        out_type=jax.ShapeDtypeStruct((B, D), jnp.float32),
        mesh=mesh,
        scratch_types=[
            pltpu.VMEM((PB * HIST,), jnp.int32),
            pltpu.VMEM((PB, D), jnp.float32),
            pltpu.VMEM((PB, D), jnp.float32),
            pltpu.VMEM((CH, D), jnp.float32),
            pltpu.VMEM((CH, D), jnp.float32),
            pltpu.SemaphoreType.DMA,
            pltpu.SemaphoreType.DMA,
        ],
    )
    uf = implicit(batch_fb.reshape(-1), item_rating_effect_weight, uw)

    bm = 256
    out = pl.pallas_call(
        _mm_body,
        grid=(B // bm,),
        in_specs=[
            pl.BlockSpec((bm, D), lambda i: (i, 0)),
            pl.BlockSpec((B, D), lambda i: (0, 0)),
            pl.BlockSpec(memory_space=pltpu.SMEM),
        ],
        out_specs=pl.BlockSpec((bm, B), lambda i: (i, 0)),
        out_shape=jax.ShapeDtypeStruct((B, B), jnp.float32),
    )(uf, iw, bias)
    return out


# speedup vs baseline: 3.2407x; 3.2407x over previous
import jax
import jax.numpy as jnp
from jax import lax
from jax.experimental import pallas as pl
from jax.experimental.pallas import tpu as pltpu
from jax.experimental.pallas import tpu_sc as plsc

NC = 2
NS = 16
NW = NC * NS
L = 16

B = 4096
D = 64
HIST = 50
PB = B // NW
CH = 128
NCH = PB * HIST // CH


def _wid():
    return lax.axis_index("s") * NC + lax.axis_index("c")


def _implicit_body(fb_flat, ew_t, uw_in, uf_o,
                   idx0, uw_v, uf_v, r0, s0):
    base = pl.multiple_of(_wid() * PB, PB)
    fb_base = pl.multiple_of(_wid() * (PB * HIST), PB * HIST)

    zero = jnp.zeros((L,), jnp.float32)

    @pl.loop(0, PB)
    def zbody(r):
        for k in range(D // L):
            uf_v[r, pl.ds(k * L, L)] = zero

    @pl.loop(0, NCH)
    def cbody(c):
        pltpu.sync_copy(fb_flat.at[pl.ds(fb_base + c * CH, CH)], idx0)
        d = pltpu.async_copy(ew_t.at[idx0], r0, s0)
        d.wait()

        @pl.loop(0, CH)
        def abody(b):
            t = (c * CH + b) // HIST
            for k in range(D // L):
                v = r0[b, pl.ds(k * L, L)]
                plsc.addupdate(uf_v.at[t, pl.ds(k * L, L)], v)

    pltpu.sync_copy(uw_in.at[pl.ds(base, PB)], uw_v)
    inv = jnp.float32(1.0 / HIST)

    @pl.loop(0, PB)
    def fbody(r):
        for k in range(D // L):
            sl = pl.ds(k * L, L)
            uf_v[r, sl] = uw_v[r, sl] + uf_v[r, sl] * inv

    pltpu.sync_copy(uf_v, uf_o.at[pl.ds(base, PB)])


def _mm_body(uf_ref, iw_ref, bias_ref, o_ref):
    acc = lax.dot_general(uf_ref[...], iw_ref[...],
                          (((1,), (1,)), ((), ())),
                          preferred_element_type=jnp.float32)
    x = acc + bias_ref[0]
    o_ref[...] = 1.0 / (1.0 + jnp.exp(-x))


def kernel(user_ids, item_ids, user_weight, item_weight,
           item_rating_effect_weight, bias, fb_indices):
    mesh = plsc.VectorSubcoreMesh(core_axis_name="c", subcore_axis_name="s",
                                  num_cores=NC, num_subcores=NS)
    sc_params = pltpu.CompilerParams(use_tc_tiling_on_sc=False)

    uw = jnp.take(user_weight, user_ids, axis=0)
    iw = jnp.take(item_weight, item_ids, axis=0)
    batch_fb = jnp.take(fb_indices, user_ids, axis=0)

    implicit = pl.kernel(
        _implicit_body,
        compiler_params=sc_params,
        out_type=jax.ShapeDtypeStruct((B, D), jnp.float32),
        mesh=mesh,
        scratch_types=[
            pltpu.VMEM((CH,), jnp.int32),
            pltpu.VMEM((PB, D), jnp.float32),
            pltpu.VMEM((PB, D), jnp.float32),
            pltpu.VMEM((CH, D), jnp.float32),
            pltpu.SemaphoreType.DMA,
        ],
    )
    uf = implicit(batch_fb.reshape(-1), item_rating_effect_weight, uw)

    bm = 256
    out = pl.pallas_call(
        _mm_body,
        grid=(B // bm,),
        in_specs=[
            pl.BlockSpec((bm, D), lambda i: (i, 0)),
            pl.BlockSpec((B, D), lambda i: (0, 0)),
            pl.BlockSpec(memory_space=pltpu.SMEM),
        ],
        out_specs=pl.BlockSpec((bm, B), lambda i: (i, 0)),
        out_shape=jax.ShapeDtypeStruct((B, B), jnp.float32),
    )(uf, iw, bias)
    return out
